# baseline (speedup 1.0000x reference)
import functools

import jax
import jax.numpy as jnp
from jax import lax
from jax.experimental import pallas as pl
from jax.experimental.pallas import tpu as pltpu
from jax.experimental.pallas import tpu_sc as plsc

DIM = 32
LANES = 16
PACK = 128 // DIM


def _make_sc_kernel(batch, vocab):
    info = plsc.get_sparse_core_info()
    nw = info.num_cores * info.num_subcores
    chunk = batch // nw
    half = chunk // 2
    mesh = plsc.VectorSubcoreMesh(core_axis_name="c", subcore_axis_name="s")

    @functools.partial(
        pl.kernel,
        mesh=mesh,
        out_type=jax.ShapeDtypeStruct((batch,), jnp.float32),
        compiler_params=pltpu.CompilerParams(
            needs_layout_passes=False,
        ),
        scratch_types=[
            pltpu.VMEM((chunk,), jnp.int32),
            pltpu.VMEM((chunk,), jnp.int32),
            pltpu.VMEM((chunk,), jnp.int32),
            pltpu.VMEM((chunk,), jnp.int32),
            pltpu.VMEM((half, 128), jnp.float32),
            pltpu.VMEM((half, 128), jnp.float32),
            pltpu.VMEM((chunk,), jnp.float32),
            pltpu.VMEM((chunk,), jnp.float32),
            pltpu.VMEM((chunk,), jnp.float32),
            pltpu.SemaphoreType.DMA,
            pltpu.SemaphoreType.DMA,
        ],
    )
    def glove_kernel(ci_hbm, xi_hbm, ctab_hbm, xtab_hbm, cb_hbm, xb_hbm,
                     out_hbm, ci_v, xi_v, cpk_v, xpk_v, crows_v, xrows_v,
                     cb_v, xb_v, out_v, sem, bsem):
        wid = lax.axis_index("s") * info.num_cores + lax.axis_index("c")
        base = pl.multiple_of(wid * chunk, chunk)

        pltpu.sync_copy(ci_hbm.at[pl.ds(base, chunk)], ci_v)
        pltpu.sync_copy(xi_hbm.at[pl.ds(base, chunk)], xi_v)

        b1 = pltpu.async_copy(cb_hbm.at[ci_v], cb_v, bsem)
        b2 = pltpu.async_copy(xb_hbm.at[xi_v], xb_v, bsem)

        def pack_body(i, carry):
            s = pl.multiple_of(i * LANES, LANES)
            cpk_v[pl.ds(s, LANES)] = ci_v[pl.ds(s, LANES)] >> 2
            xpk_v[pl.ds(s, LANES)] = xi_v[pl.ds(s, LANES)] >> 2
            return carry

        lax.fori_loop(0, chunk // LANES, pack_body, 0)

        iota = lax.iota(jnp.int32, LANES)
        b1.wait()
        b2.wait()

        for h in range(2):
            hbase = h * half
            g1 = pltpu.async_copy(
                ctab_hbm.at[cpk_v.at[pl.ds(hbase, half)]], crows_v, sem)
            g2 = pltpu.async_copy(
                xtab_hbm.at[xpk_v.at[pl.ds(hbase, half)]], xrows_v, sem)
            g1.wait()
            g2.wait()

            def blk_body(blk, carry):
                lb = pl.multiple_of(blk * LANES, LANES)
                gb = hbase + lb
                rows = lb + iota
                ci16 = ci_v[pl.ds(gb, LANES)]
                xi16 = xi_v[pl.ds(gb, LANES)]
                coff = (ci16 & 3) * DIM
                xoff = (xi16 & 3) * DIM
                acc = cb_v[pl.ds(gb, LANES)] + xb_v[pl.ds(gb, LANES)]
                for d in range(DIM):
                    acc = acc + (plsc.load_gather(crows_v, [rows, coff + d]) *
                                 plsc.load_gather(xrows_v, [rows, xoff + d]))
                out_v[pl.ds(gb, LANES)] = acc
                return carry

            lax.fori_loop(0, half // LANES, blk_body, 0)

        pltpu.sync_copy(out_v, out_hbm.at[pl.ds(base, chunk)])

    return glove_kernel


def kernel(center_word_idx, context_word_idx, center_embeddings,
           context_embeddings, center_biases, context_biases):
    batch = center_word_idx.shape[0]
    vocab = center_embeddings.shape[0]
    ci = center_word_idx.astype(jnp.int32)
    xi = context_word_idx.astype(jnp.int32)
    ctab = center_embeddings.reshape(vocab // PACK, 128)
    xtab = context_embeddings.reshape(vocab // PACK, 128)
    cb = center_biases.reshape(vocab)
    xb = context_biases.reshape(vocab)
    sc_kernel = _make_sc_kernel(batch, vocab)
    return sc_kernel(ci, xi, ctab, xtab, cb, xb)
